# Initial kernel scaffold; baseline (speedup 1.0000x reference)
import functools

import jax
import jax.numpy as jnp
from jax import lax
from jax.experimental import pallas as pl
from jax.experimental.pallas import tpu as pltpu
from jax.experimental.pallas import tpu_sc as plsc

N = 10000
E = 160000
F_IN = 128
H = 32
C = 40

NC = 2
NS = 16
NW = NC * NS
CH = 128
NROWS = E // CH
BASE_ROWS = NROWS // NW
EXTRA = NROWS - BASE_ROWS * NW
MAXR = BASE_ROWS + 1
PAD_ROWS = MAXR * NW

_MESH = plsc.VectorSubcoreMesh(core_axis_name="c", subcore_axis_name="s")


@functools.partial(
    pl.kernel,
    out_type=jax.ShapeDtypeStruct((NC, N), jnp.float32),
    mesh=_MESH,
    scratch_types=[
        pltpu.VMEM((MAXR, CH), jnp.int32),
        pltpu.VMEM((CH,), jnp.float32),
        pltpu.VMEM_SHARED((N,), jnp.float32),
        pltpu.SemaphoreType.DMA,
    ],
)
def _deg_sc(dst2d_hbm, ones_hbm, zeros_hbm, out_hbm, dsti_v, ones_v, acc_sh, sem):
    c = lax.axis_index("c")
    s = lax.axis_index("s")
    w = c * NS + s
    start_row = w * BASE_ROWS + jnp.minimum(w, EXTRA)
    nrows = BASE_ROWS + jnp.where(w < EXTRA, 1, 0)

    @pl.when(s == 0)
    def _zero():
        pltpu.sync_copy(zeros_hbm, acc_sh)

    pltpu.sync_copy(dst2d_hbm.at[pl.ds(start_row, MAXR)], dsti_v)
    pltpu.sync_copy(ones_hbm, ones_v)
    plsc.subcore_barrier()

    def body(j, carry):
        pltpu.sync_copy(ones_v, acc_sh.at[dsti_v.at[j]], add=True)
        return carry

    lax.fori_loop(0, nrows, body, 0)
    plsc.subcore_barrier()

    @pl.when(s == 0)
    def _flush():
        pltpu.sync_copy(acc_sh, out_hbm.at[c])


@functools.partial(
    pl.kernel,
    out_type=jax.ShapeDtypeStruct((NC, N, H), jnp.float32),
    mesh=_MESH,
    scratch_types=[
        pltpu.VMEM((MAXR * CH,), jnp.int32),
        pltpu.VMEM((MAXR, CH), jnp.int32),
        pltpu.VMEM((CH, H), jnp.float32),
        pltpu.VMEM_SHARED((N, H), jnp.float32),
        pltpu.SemaphoreType.DMA,
    ],
)
def _agg_sc(hp_hbm, src_hbm, dst2d_hbm, zeros_hbm, out_hbm,
            srci_v, dsti_v, rows_v, acc_sh, sem):
    c = lax.axis_index("c")
    s = lax.axis_index("s")
    w = c * NS + s
    start_row = w * BASE_ROWS + jnp.minimum(w, EXTRA)
    nrows = BASE_ROWS + jnp.where(w < EXTRA, 1, 0)

    @pl.when(s == 0)
    def _zero():
        pltpu.sync_copy(zeros_hbm, acc_sh)

    pltpu.sync_copy(src_hbm.at[pl.ds(start_row * CH, MAXR * CH)], srci_v)
    pltpu.sync_copy(dst2d_hbm.at[pl.ds(start_row, MAXR)], dsti_v)
    plsc.subcore_barrier()

    def body(j, carry):
        pltpu.async_copy(hp_hbm.at[srci_v.at[pl.ds(j * CH, CH)]], rows_v, sem).wait()
        pltpu.sync_copy(rows_v, acc_sh.at[dsti_v.at[j]], add=True)
        return carry

    lax.fori_loop(0, nrows, body, 0)
    plsc.subcore_barrier()

    @pl.when(s == 0)
    def _flush():
        pltpu.sync_copy(acc_sh, out_hbm.at[c])


_RB = 200


def _fold_body(b_ref, a1w_ref, a1b_ref, a2w_ref, a2b_ref, a3w_ref, a3b_ref,
               a4w_ref, a4b_ref, bae1_ref, bout_ref):
    v1 = jnp.dot(b_ref[...], a1w_ref[...], preferred_element_type=jnp.float32)
    v1 = v1 + a1b_ref[...]
    bae1_ref[...] = v1
    u = jnp.dot(v1, a2w_ref[...], preferred_element_type=jnp.float32) + a2b_ref[...]
    u = jnp.dot(u, a3w_ref[...], preferred_element_type=jnp.float32) + a3b_ref[...]
    bout_ref[...] = (
        jnp.dot(u, a4w_ref[...], preferred_element_type=jnp.float32) + a4b_ref[...]
    )


def _fold(b, A1W, A1b, A2W, A2b, A3W, A3b, A4W, A4b):
    full = lambda shape: pl.BlockSpec(shape, lambda i: (0, 0))
    return pl.pallas_call(
        _fold_body,
        grid=(N // _RB,),
        in_specs=[
            pl.BlockSpec((_RB, N), lambda i: (i, 0)),
            full((N, H)), full((1, H)),
            full((H, H)), full((1, H)),
            full((H, H)), full((1, H)),
            full((H, N)), full((1, N)),
        ],
        out_specs=[
            pl.BlockSpec((_RB, H), lambda i: (i, 0)),
            pl.BlockSpec((_RB, N), lambda i: (i, 0)),
        ],
        out_shape=[
            jax.ShapeDtypeStruct((N, H), jnp.float32),
            jax.ShapeDtypeStruct((N, N), jnp.float32),
        ],
        compiler_params=pltpu.CompilerParams(
            dimension_semantics=("parallel",),
        ),
    )(b, A1W, A1b.reshape(1, H), A2W, A2b.reshape(1, H),
      A3W, A3b.reshape(1, H), A4W, A4b.reshape(1, N))


def _glue1_body(x_ref, w1_ref, degc_ref, hp1_ref):
    dinv = lax.rsqrt(degc_ref[...] + 1.0)
    hp1_ref[...] = (
        jnp.dot(x_ref[...], w1_ref[...], preferred_element_type=jnp.float32) * dinv
    )


def _glue1(x, W1, degcol):
    return pl.pallas_call(
        _glue1_body,
        out_shape=jax.ShapeDtypeStruct((N, H), jnp.float32),
    )(x, W1, degcol)


def _bn_relu(t, g, be):
    m = jnp.mean(t, axis=0, keepdims=True)
    v = jnp.mean((t - m) ** 2, axis=0, keepdims=True)
    return jnp.maximum((t - m) * lax.rsqrt(v + 1e-5) * g + be, 0.0)


def _glue2_body(a_ref, b_ref, hp1_ref, degc_ref, b1_ref, g1_ref, be1_ref,
                bae1_ref, w2_ref, x0_ref, hp2_ref):
    dinv = lax.rsqrt(degc_ref[...] + 1.0)
    t = dinv * (a_ref[...] + b_ref[...] + hp1_ref[...]) + b1_ref[...]
    x0 = _bn_relu(t, g1_ref[...], be1_ref[...])
    x0_ref[...] = x0
    hp2_ref[...] = (
        jnp.dot(x0 + bae1_ref[...], w2_ref[...], preferred_element_type=jnp.float32)
        * dinv
    )


def _glue2(agg_a, agg_b, hp1, degcol, b1, g1, be1, bae1, W2):
    return pl.pallas_call(
        _glue2_body,
        out_shape=[
            jax.ShapeDtypeStruct((N, H), jnp.float32),
            jax.ShapeDtypeStruct((N, H), jnp.float32),
        ],
    )(agg_a, agg_b, hp1, degcol, b1.reshape(1, H), g1.reshape(1, H),
      be1.reshape(1, H), bae1, W2)


def _glue3_body(a_ref, b_ref, hp2_ref, degc_ref, b2_ref, g2_ref, be2_ref,
                x0_ref, jwa_ref, jwb_ref, jb_ref, out_ref):
    dinv = lax.rsqrt(degc_ref[...] + 1.0)
    t = dinv * (a_ref[...] + b_ref[...] + hp2_ref[...]) + b2_ref[...]
    x1 = _bn_relu(t, g2_ref[...], be2_ref[...])
    logits = (
        jnp.dot(x0_ref[...], jwa_ref[...], preferred_element_type=jnp.float32)
        + jnp.dot(x1, jwb_ref[...], preferred_element_type=jnp.float32)
        + jb_ref[...]
    )
    mx = jnp.max(logits, axis=1, keepdims=True)
    sh = logits - mx
    out_ref[...] = sh - jnp.log(jnp.sum(jnp.exp(sh), axis=1, keepdims=True))


def _glue3(agg_a, agg_b, hp2, degcol, b2, g2, be2, x0, JW, Jb):
    return pl.pallas_call(
        _glue3_body,
        out_shape=jax.ShapeDtypeStruct((N, C), jnp.float32),
    )(agg_a, agg_b, hp2, degcol, b2.reshape(1, H), g2.reshape(1, H),
      be2.reshape(1, H), x0, JW[:H], JW[H:], Jb.reshape(1, C))


def kernel(b, x, edge_index, W1, b1, W2, b2, g1, be1, g2, be2,
           A1W, A1b, A2W, A2b, A3W, A3b, A4W, A4b, JW, Jb):
    src = edge_index[0]
    dst = edge_index[1]
    pad = PAD_ROWS * CH - E
    zpad = jnp.zeros((pad,), jnp.int32)
    src_pad = jnp.concatenate([src, zpad])
    dst2d = jnp.concatenate([dst, zpad]).reshape(PAD_ROWS, CH)
    zeros_n = jnp.zeros((N,), jnp.float32)
    zeros_nh = jnp.zeros((N, H), jnp.float32)
    ones_ch = jnp.ones((CH,), jnp.float32)

    deg2 = _deg_sc(dst2d, ones_ch, zeros_n)
    degcol = (deg2[0] + deg2[1]).reshape(N, 1)

    hp1 = _glue1(x, W1, degcol)
    agg1 = _agg_sc(hp1, src_pad, dst2d, zeros_nh)

    bae1, bout = _fold(b, A1W, A1b, A2W, A2b, A3W, A3b, A4W, A4b)

    x0, hp2 = _glue2(agg1[0], agg1[1], hp1, degcol, b1, g1, be1, bae1, W2)
    agg2 = _agg_sc(hp2, src_pad, dst2d, zeros_nh)
    out1 = _glue3(agg2[0], agg2[1], hp2, degcol, b2, g2, be2, x0, JW, Jb)
    return out1, bout


# speedup vs baseline: 11.3394x; 11.3394x over previous
import functools

import jax
import jax.numpy as jnp
from jax import lax
from jax.experimental import pallas as pl
from jax.experimental.pallas import tpu as pltpu
from jax.experimental.pallas import tpu_sc as plsc

N = 10000
E = 160000
F_IN = 128
H = 32
C = 40

NC = 2
NS = 16
NW = NC * NS
CH = 128
NROWS = E // CH
MAXR = (NROWS + NW - 1) // NW
PAD_ROWS = MAXR * NW
NP = N + 16
NPD = 10240

_MESH = plsc.VectorSubcoreMesh(core_axis_name="c", subcore_axis_name="s")


@functools.partial(
    pl.kernel,
    out_type=jax.ShapeDtypeStruct((NC, NPD), jnp.float32),
    mesh=_MESH,
    scratch_types=[
        pltpu.VMEM((MAXR, CH), jnp.int32),
        pltpu.VMEM((CH,), jnp.float32),
        pltpu.VMEM_SHARED((NPD,), jnp.float32),
        pltpu.SemaphoreType.DMA,
    ],
    compiler_params=pltpu.CompilerParams(use_tc_tiling_on_sc=False),
)
def _deg_sc(dst2d_hbm, ones_hbm, zeros_hbm, out_hbm, dsti_v, ones_v, acc_sh, sem):
    c = lax.axis_index("c")
    s = lax.axis_index("s")
    w = c * NS + s
    start_row = w * MAXR

    @pl.when(s == 0)
    def _zero():
        pltpu.sync_copy(zeros_hbm, acc_sh)

    pltpu.sync_copy(dst2d_hbm.at[pl.ds(start_row, MAXR)], dsti_v)
    pltpu.sync_copy(ones_hbm, ones_v)
    plsc.subcore_barrier()

    def body(j, carry):
        pltpu.sync_copy(ones_v, acc_sh.at[dsti_v.at[j]], add=True)
        return carry

    lax.fori_loop(0, MAXR, body, 0)
    plsc.subcore_barrier()

    @pl.when(s == 0)
    def _flush():
        pltpu.sync_copy(acc_sh, out_hbm.at[c])


@functools.partial(
    pl.kernel,
    out_type=jax.ShapeDtypeStruct((NC, N, H), jnp.float32),
    mesh=_MESH,
    scratch_types=[
        pltpu.VMEM((MAXR * CH,), jnp.int32),
        pltpu.VMEM((MAXR, CH), jnp.int32),
        pltpu.VMEM((CH, H), jnp.float32),
        pltpu.VMEM_SHARED((N, H), jnp.float32),
        pltpu.VMEM_SHARED((NP, H), jnp.float32),
        pltpu.SemaphoreType.DMA,
    ],
    compiler_params=pltpu.CompilerParams(use_tc_tiling_on_sc=False),
)
def _agg_sc(hp_hbm, src_hbm, dst2d_hbm, zeros_hbm, out_hbm,
            srci_v, dsti_v, rows_v, hp_sh, acc_sh, sem):
    c = lax.axis_index("c")
    s = lax.axis_index("s")
    w = c * NS + s
    start_row = w * MAXR

    TR = N // NS
    ZR = NP // NS
    pltpu.sync_copy(hp_hbm.at[pl.ds(s * TR, TR)], hp_sh.at[pl.ds(s * TR, TR)])
    pltpu.sync_copy(zeros_hbm.at[pl.ds(s * ZR, ZR)], acc_sh.at[pl.ds(s * ZR, ZR)])

    pltpu.sync_copy(src_hbm.at[pl.ds(start_row * CH, MAXR * CH)], srci_v)
    pltpu.sync_copy(dst2d_hbm.at[pl.ds(start_row, MAXR)], dsti_v)
    plsc.subcore_barrier()

    def body(j, carry):
        pltpu.async_copy(hp_sh.at[srci_v.at[pl.ds(j * CH, CH)]], rows_v, sem).wait()
        pltpu.sync_copy(rows_v, acc_sh.at[dsti_v.at[j]], add=True)
        return carry

    lax.fori_loop(0, MAXR, body, 0)
    plsc.subcore_barrier()

    @pl.when(s == 0)
    def _flush():
        pltpu.sync_copy(acc_sh.at[pl.ds(0, N)], out_hbm.at[c])


_RB = 200


def _fold_body(b_ref, a1w_ref, a1b_ref, a2w_ref, a2b_ref, a3w_ref, a3b_ref,
               a4w_ref, a4b_ref, bae1_ref, bout_ref):
    v1 = jnp.dot(b_ref[...], a1w_ref[...], preferred_element_type=jnp.float32)
    v1 = v1 + a1b_ref[...]
    bae1_ref[...] = v1
    u = jnp.dot(v1, a2w_ref[...], preferred_element_type=jnp.float32) + a2b_ref[...]
    u = jnp.dot(u, a3w_ref[...], preferred_element_type=jnp.float32) + a3b_ref[...]
    bout_ref[...] = (
        jnp.dot(u, a4w_ref[...], preferred_element_type=jnp.float32) + a4b_ref[...]
    )


def _fold(b, A1W, A1b, A2W, A2b, A3W, A3b, A4W, A4b):
    full = lambda shape: pl.BlockSpec(shape, lambda i: (0, 0))
    return pl.pallas_call(
        _fold_body,
        grid=(N // _RB,),
        in_specs=[
            pl.BlockSpec((_RB, N), lambda i: (i, 0)),
            full((N, H)), full((1, H)),
            full((H, H)), full((1, H)),
            full((H, H)), full((1, H)),
            full((H, N)), full((1, N)),
        ],
        out_specs=[
            pl.BlockSpec((_RB, H), lambda i: (i, 0)),
            pl.BlockSpec((_RB, N), lambda i: (i, 0)),
        ],
        out_shape=[
            jax.ShapeDtypeStruct((N, H), jnp.float32),
            jax.ShapeDtypeStruct((N, N), jnp.float32),
        ],
        compiler_params=pltpu.CompilerParams(
            dimension_semantics=("parallel",),
        ),
    )(b, A1W, A1b.reshape(1, H), A2W, A2b.reshape(1, H),
      A3W, A3b.reshape(1, H), A4W, A4b.reshape(1, N))


def _glue1_body(x_ref, w1_ref, degc_ref, hp1_ref):
    dinv = lax.rsqrt(degc_ref[...] + 1.0)
    hp1_ref[...] = (
        jnp.dot(x_ref[...], w1_ref[...], preferred_element_type=jnp.float32) * dinv
    )


def _glue1(x, W1, degcol):
    return pl.pallas_call(
        _glue1_body,
        out_shape=jax.ShapeDtypeStruct((N, H), jnp.float32),
    )(x, W1, degcol)


def _bn_relu(t, g, be):
    m = jnp.mean(t, axis=0, keepdims=True)
    v = jnp.mean((t - m) ** 2, axis=0, keepdims=True)
    return jnp.maximum((t - m) * lax.rsqrt(v + 1e-5) * g + be, 0.0)


def _glue2_body(a_ref, b_ref, hp1_ref, degc_ref, b1_ref, g1_ref, be1_ref,
                bae1_ref, w2_ref, x0_ref, hp2_ref):
    dinv = lax.rsqrt(degc_ref[...] + 1.0)
    t = dinv * (a_ref[...] + b_ref[...] + hp1_ref[...]) + b1_ref[...]
    x0 = _bn_relu(t, g1_ref[...], be1_ref[...])
    x0_ref[...] = x0
    hp2_ref[...] = (
        jnp.dot(x0 + bae1_ref[...], w2_ref[...], preferred_element_type=jnp.float32)
        * dinv
    )


def _glue2(agg_a, agg_b, hp1, degcol, b1, g1, be1, bae1, W2):
    return pl.pallas_call(
        _glue2_body,
        out_shape=[
            jax.ShapeDtypeStruct((N, H), jnp.float32),
            jax.ShapeDtypeStruct((N, H), jnp.float32),
        ],
    )(agg_a, agg_b, hp1, degcol, b1.reshape(1, H), g1.reshape(1, H),
      be1.reshape(1, H), bae1, W2)


def _glue3_body(a_ref, b_ref, hp2_ref, degc_ref, b2_ref, g2_ref, be2_ref,
                x0_ref, jwa_ref, jwb_ref, jb_ref, out_ref):
    dinv = lax.rsqrt(degc_ref[...] + 1.0)
    t = dinv * (a_ref[...] + b_ref[...] + hp2_ref[...]) + b2_ref[...]
    x1 = _bn_relu(t, g2_ref[...], be2_ref[...])
    logits = (
        jnp.dot(x0_ref[...], jwa_ref[...], preferred_element_type=jnp.float32)
        + jnp.dot(x1, jwb_ref[...], preferred_element_type=jnp.float32)
        + jb_ref[...]
    )
    mx = jnp.max(logits, axis=1, keepdims=True)
    sh = logits - mx
    out_ref[...] = sh - jnp.log(jnp.sum(jnp.exp(sh), axis=1, keepdims=True))


def _glue3(agg_a, agg_b, hp2, degcol, b2, g2, be2, x0, JW, Jb):
    return pl.pallas_call(
        _glue3_body,
        out_shape=jax.ShapeDtypeStruct((N, C), jnp.float32),
    )(agg_a, agg_b, hp2, degcol, b2.reshape(1, H), g2.reshape(1, H),
      be2.reshape(1, H), x0, JW[:H], JW[H:], Jb.reshape(1, C))


def kernel(b, x, edge_index, W1, b1, W2, b2, g1, be1, g2, be2,
           A1W, A1b, A2W, A2b, A3W, A3b, A4W, A4b, JW, Jb):
    src = edge_index[0]
    dst = edge_index[1]
    pad = PAD_ROWS * CH - E
    src_pad = jnp.concatenate([src, jnp.zeros((pad,), jnp.int32)])
    dpad = N + (jnp.arange(pad, dtype=jnp.int32) % 8)
    dst2d = jnp.concatenate([dst, dpad]).reshape(PAD_ROWS, CH)
    zeros_n = jnp.zeros((NPD,), jnp.float32)
    zeros_nh = jnp.zeros((NP, H), jnp.float32)
    ones_ch = jnp.ones((CH,), jnp.float32)

    deg2 = _deg_sc(dst2d, ones_ch, zeros_n)
    degcol = (deg2[0, :N] + deg2[1, :N]).reshape(N, 1)

    hp1 = _glue1(x, W1, degcol)
    agg1 = _agg_sc(hp1, src_pad, dst2d, zeros_nh)

    bae1, bout = _fold(b, A1W, A1b, A2W, A2b, A3W, A3b, A4W, A4b)

    x0, hp2 = _glue2(agg1[0], agg1[1], hp1, degcol, b1, g1, be1, bae1, W2)
    agg2 = _agg_sc(hp2, src_pad, dst2d, zeros_nh)
    out1 = _glue3(agg2[0], agg2[1], hp2, degcol, b2, g2, be2, x0, JW, Jb)
    return out1, bout


# speedup vs baseline: 11.5018x; 1.0143x over previous
import functools

import jax
import jax.numpy as jnp
from jax import lax
from jax.experimental import pallas as pl
from jax.experimental.pallas import tpu as pltpu
from jax.experimental.pallas import tpu_sc as plsc

N = 10000
E = 160000
F_IN = 128
H = 32
C = 40

NC = 2
NS = 16
NW = NC * NS
CH = 128
NROWS = E // CH
MAXR = (NROWS + NW - 1) // NW
PAD_ROWS = MAXR * NW
NP = N + 16
NPD = 10240

_MESH = plsc.VectorSubcoreMesh(core_axis_name="c", subcore_axis_name="s")


@functools.partial(
    pl.kernel,
    out_type=jax.ShapeDtypeStruct((NC, NPD), jnp.float32),
    mesh=_MESH,
    scratch_types=[
        pltpu.VMEM((MAXR, CH), jnp.int32),
        pltpu.VMEM((CH,), jnp.float32),
        pltpu.VMEM_SHARED((NPD,), jnp.float32),
        pltpu.SemaphoreType.DMA,
    ],
    compiler_params=pltpu.CompilerParams(use_tc_tiling_on_sc=False),
)
def _deg_sc(dst2d_hbm, ones_hbm, zeros_hbm, out_hbm, dsti_v, ones_v, acc_sh, sem):
    c = lax.axis_index("c")
    s = lax.axis_index("s")
    w = c * NS + s
    start_row = w * MAXR

    @pl.when(s == 0)
    def _zero():
        pltpu.sync_copy(zeros_hbm, acc_sh)

    pltpu.sync_copy(dst2d_hbm.at[pl.ds(start_row, MAXR)], dsti_v)
    pltpu.sync_copy(ones_hbm, ones_v)
    plsc.subcore_barrier()

    def body(j, carry):
        pltpu.sync_copy(ones_v, acc_sh.at[dsti_v.at[j]], add=True)
        return carry

    lax.fori_loop(0, MAXR, body, 0)
    plsc.subcore_barrier()

    @pl.when(s == 0)
    def _flush():
        pltpu.sync_copy(acc_sh, out_hbm.at[c])


@functools.partial(
    pl.kernel,
    out_type=jax.ShapeDtypeStruct((NC, N, H), jnp.float32),
    mesh=_MESH,
    scratch_types=[
        pltpu.VMEM((MAXR * CH,), jnp.int32),
        pltpu.VMEM((MAXR, CH), jnp.int32),
        pltpu.VMEM((CH, H), jnp.float32),
        pltpu.VMEM((CH, H), jnp.float32),
        pltpu.VMEM_SHARED((N, H), jnp.float32),
        pltpu.VMEM_SHARED((NP, H), jnp.float32),
        pltpu.SemaphoreType.DMA,
        pltpu.SemaphoreType.DMA,
    ],
    compiler_params=pltpu.CompilerParams(use_tc_tiling_on_sc=False),
)
def _agg_sc(hp_hbm, src_hbm, dst2d_hbm, zeros_hbm, out_hbm,
            srci_v, dsti_v, rows0_v, rows1_v, hp_sh, acc_sh, sem0, sem1):
    c = lax.axis_index("c")
    s = lax.axis_index("s")
    w = c * NS + s
    start_row = w * MAXR

    TR = N // NS
    ZR = NP // NS
    pltpu.sync_copy(hp_hbm.at[pl.ds(s * TR, TR)], hp_sh.at[pl.ds(s * TR, TR)])
    pltpu.sync_copy(zeros_hbm.at[pl.ds(s * ZR, ZR)], acc_sh.at[pl.ds(s * ZR, ZR)])

    pltpu.sync_copy(src_hbm.at[pl.ds(start_row * CH, MAXR * CH)], srci_v)
    pltpu.sync_copy(dst2d_hbm.at[pl.ds(start_row, MAXR)], dsti_v)
    plsc.subcore_barrier()

    pltpu.async_copy(hp_sh.at[srci_v.at[pl.ds(0, CH)]], rows0_v, sem0)

    def body(g, carry):
        j0 = 2 * g
        j1 = j0 + 1
        cp1 = pltpu.async_copy(
            hp_sh.at[srci_v.at[pl.ds(j1 * CH, CH)]], rows1_v, sem1)
        pltpu.make_async_copy(hp_hbm.at[pl.ds(0, CH)], rows0_v, sem0).wait()
        pltpu.sync_copy(rows0_v, acc_sh.at[dsti_v.at[j0]], add=True)
        nxt = j0 + 2

        @pl.when(nxt < MAXR)
        def _issue_next_even():
            pltpu.async_copy(
                hp_sh.at[srci_v.at[pl.ds(nxt * CH, CH)]], rows0_v, sem0)

        cp1.wait()
        pltpu.sync_copy(rows1_v, acc_sh.at[dsti_v.at[j1]], add=True)
        return carry

    lax.fori_loop(0, MAXR // 2, body, 0)
    plsc.subcore_barrier()

    @pl.when(s == 0)
    def _flush():
        pltpu.sync_copy(acc_sh.at[pl.ds(0, N)], out_hbm.at[c])


_RB = 200


def _fold_body(b_ref, a1w_ref, a1b_ref, a2w_ref, a2b_ref, a3w_ref, a3b_ref,
               a4w_ref, a4b_ref, bae1_ref, bout_ref):
    v1 = jnp.dot(b_ref[...], a1w_ref[...], preferred_element_type=jnp.float32)
    v1 = v1 + a1b_ref[...]
    bae1_ref[...] = v1
    u = jnp.dot(v1, a2w_ref[...], preferred_element_type=jnp.float32) + a2b_ref[...]
    u = jnp.dot(u, a3w_ref[...], preferred_element_type=jnp.float32) + a3b_ref[...]
    bout_ref[...] = (
        jnp.dot(u, a4w_ref[...], preferred_element_type=jnp.float32) + a4b_ref[...]
    )


def _fold(b, A1W, A1b, A2W, A2b, A3W, A3b, A4W, A4b):
    full = lambda shape: pl.BlockSpec(shape, lambda i: (0, 0))
    return pl.pallas_call(
        _fold_body,
        grid=(N // _RB,),
        in_specs=[
            pl.BlockSpec((_RB, N), lambda i: (i, 0)),
            full((N, H)), full((1, H)),
            full((H, H)), full((1, H)),
            full((H, H)), full((1, H)),
            full((H, N)), full((1, N)),
        ],
        out_specs=[
            pl.BlockSpec((_RB, H), lambda i: (i, 0)),
            pl.BlockSpec((_RB, N), lambda i: (i, 0)),
        ],
        out_shape=[
            jax.ShapeDtypeStruct((N, H), jnp.float32),
            jax.ShapeDtypeStruct((N, N), jnp.float32),
        ],
        compiler_params=pltpu.CompilerParams(
            dimension_semantics=("parallel",),
        ),
    )(b, A1W, A1b.reshape(1, H), A2W, A2b.reshape(1, H),
      A3W, A3b.reshape(1, H), A4W, A4b.reshape(1, N))


def _glue1_body(x_ref, w1_ref, degc_ref, hp1_ref):
    dinv = lax.rsqrt(degc_ref[...] + 1.0)
    hp1_ref[...] = (
        jnp.dot(x_ref[...], w1_ref[...], preferred_element_type=jnp.float32) * dinv
    )


def _glue1(x, W1, degcol):
    return pl.pallas_call(
        _glue1_body,
        out_shape=jax.ShapeDtypeStruct((N, H), jnp.float32),
    )(x, W1, degcol)


def _bn_relu(t, g, be):
    m = jnp.mean(t, axis=0, keepdims=True)
    v = jnp.mean((t - m) ** 2, axis=0, keepdims=True)
    return jnp.maximum((t - m) * lax.rsqrt(v + 1e-5) * g + be, 0.0)


def _glue2_body(a_ref, b_ref, hp1_ref, degc_ref, b1_ref, g1_ref, be1_ref,
                bae1_ref, w2_ref, x0_ref, hp2_ref):
    dinv = lax.rsqrt(degc_ref[...] + 1.0)
    t = dinv * (a_ref[...] + b_ref[...] + hp1_ref[...]) + b1_ref[...]
    x0 = _bn_relu(t, g1_ref[...], be1_ref[...])
    x0_ref[...] = x0
    hp2_ref[...] = (
        jnp.dot(x0 + bae1_ref[...], w2_ref[...], preferred_element_type=jnp.float32)
        * dinv
    )


def _glue2(agg_a, agg_b, hp1, degcol, b1, g1, be1, bae1, W2):
    return pl.pallas_call(
        _glue2_body,
        out_shape=[
            jax.ShapeDtypeStruct((N, H), jnp.float32),
            jax.ShapeDtypeStruct((N, H), jnp.float32),
        ],
    )(agg_a, agg_b, hp1, degcol, b1.reshape(1, H), g1.reshape(1, H),
      be1.reshape(1, H), bae1, W2)


def _glue3_body(a_ref, b_ref, hp2_ref, degc_ref, b2_ref, g2_ref, be2_ref,
                x0_ref, jwa_ref, jwb_ref, jb_ref, out_ref):
    dinv = lax.rsqrt(degc_ref[...] + 1.0)
    t = dinv * (a_ref[...] + b_ref[...] + hp2_ref[...]) + b2_ref[...]
    x1 = _bn_relu(t, g2_ref[...], be2_ref[...])
    logits = (
        jnp.dot(x0_ref[...], jwa_ref[...], preferred_element_type=jnp.float32)
        + jnp.dot(x1, jwb_ref[...], preferred_element_type=jnp.float32)
        + jb_ref[...]
    )
    mx = jnp.max(logits, axis=1, keepdims=True)
    sh = logits - mx
    out_ref[...] = sh - jnp.log(jnp.sum(jnp.exp(sh), axis=1, keepdims=True))


def _glue3(agg_a, agg_b, hp2, degcol, b2, g2, be2, x0, JW, Jb):
    return pl.pallas_call(
        _glue3_body,
        out_shape=jax.ShapeDtypeStruct((N, C), jnp.float32),
    )(agg_a, agg_b, hp2, degcol, b2.reshape(1, H), g2.reshape(1, H),
      be2.reshape(1, H), x0, JW[:H], JW[H:], Jb.reshape(1, C))


def kernel(b, x, edge_index, W1, b1, W2, b2, g1, be1, g2, be2,
           A1W, A1b, A2W, A2b, A3W, A3b, A4W, A4b, JW, Jb):
    src = edge_index[0]
    dst = edge_index[1]
    pad = PAD_ROWS * CH - E
    src_pad = jnp.concatenate([src, jnp.zeros((pad,), jnp.int32)])
    dpad = N + (jnp.arange(pad, dtype=jnp.int32) % 8)
    dst2d = jnp.concatenate([dst, dpad]).reshape(PAD_ROWS, CH)
    zeros_n = jnp.zeros((NPD,), jnp.float32)
    zeros_nh = jnp.zeros((NP, H), jnp.float32)
    ones_ch = jnp.ones((CH,), jnp.float32)

    deg2 = _deg_sc(dst2d, ones_ch, zeros_n)
    degcol = (deg2[0, :N] + deg2[1, :N]).reshape(N, 1)

    hp1 = _glue1(x, W1, degcol)
    agg1 = _agg_sc(hp1, src_pad, dst2d, zeros_nh)

    bae1, bout = _fold(b, A1W, A1b, A2W, A2b, A3W, A3b, A4W, A4b)

    x0, hp2 = _glue2(agg1[0], agg1[1], hp1, degcol, b1, g1, be1, bae1, W2)
    agg2 = _agg_sc(hp2, src_pad, dst2d, zeros_nh)
    out1 = _glue3(agg2[0], agg2[1], hp2, degcol, b2, g2, be2, x0, JW, Jb)
    return out1, bout


# speedup vs baseline: 13.0110x; 1.1312x over previous
import functools

import jax
import jax.numpy as jnp
from jax import lax
from jax.experimental import pallas as pl
from jax.experimental.pallas import tpu as pltpu
from jax.experimental.pallas import tpu_sc as plsc

N = 10000
E = 160000
F_IN = 128
H = 32
C = 40

NC = 2
NS = 16
NW = NC * NS
CH = 128
NROWS = E // CH
MAXR = (NROWS + NW - 1) // NW
PAD_ROWS = MAXR * NW
NP = N + 16
NPD = 10240

_MESH = plsc.VectorSubcoreMesh(core_axis_name="c", subcore_axis_name="s")


@functools.partial(
    pl.kernel,
    out_type=jax.ShapeDtypeStruct((NC, NPD), jnp.float32),
    mesh=_MESH,
    scratch_types=[
        pltpu.VMEM((MAXR, CH), jnp.int32),
        pltpu.VMEM((CH,), jnp.float32),
        pltpu.VMEM_SHARED((NPD,), jnp.float32),
        pltpu.SemaphoreType.DMA,
    ],
    compiler_params=pltpu.CompilerParams(use_tc_tiling_on_sc=False),
)
def _deg_sc(dst2d_hbm, ones_hbm, zeros_hbm, out_hbm, dsti_v, ones_v, acc_sh, sem):
    c = lax.axis_index("c")
    s = lax.axis_index("s")
    w = c * NS + s
    start_row = w * MAXR

    @pl.when(s == 0)
    def _zero():
        pltpu.sync_copy(zeros_hbm, acc_sh)

    pltpu.sync_copy(dst2d_hbm.at[pl.ds(start_row, MAXR)], dsti_v)
    pltpu.sync_copy(ones_hbm, ones_v)
    plsc.subcore_barrier()

    def body(j, carry):
        pltpu.sync_copy(ones_v, acc_sh.at[dsti_v.at[j]], add=True)
        return carry

    lax.fori_loop(0, MAXR, body, 0)
    plsc.subcore_barrier()

    @pl.when(s == 0)
    def _flush():
        pltpu.sync_copy(acc_sh, out_hbm.at[c])


@functools.partial(
    pl.kernel,
    out_type=jax.ShapeDtypeStruct((NC, N, H), jnp.float32),
    mesh=_MESH,
    scratch_types=[
        pltpu.VMEM((MAXR * CH,), jnp.int32),
        pltpu.VMEM((MAXR, CH), jnp.int32),
        pltpu.VMEM((CH, H), jnp.float32),
        pltpu.VMEM((CH, H), jnp.float32),
        pltpu.VMEM_SHARED((N, H), jnp.float32),
        pltpu.VMEM_SHARED((NP, H), jnp.float32),
        pltpu.SemaphoreType.DMA,
        pltpu.SemaphoreType.DMA,
    ],
    compiler_params=pltpu.CompilerParams(use_tc_tiling_on_sc=False),
)
def _agg_sc(hp_hbm, src_hbm, dst2d_hbm, zeros_hbm, out_hbm,
            srci_v, dsti_v, rows0_v, rows1_v, hp_sh, acc_sh, sem0, sem1):
    c = lax.axis_index("c")
    s = lax.axis_index("s")
    w = c * NS + s
    start_row = w * MAXR

    TR = N // NS
    ZR = NP // NS
    pltpu.sync_copy(hp_hbm.at[pl.ds(s * TR, TR)], hp_sh.at[pl.ds(s * TR, TR)])
    pltpu.sync_copy(zeros_hbm.at[pl.ds(s * ZR, ZR)], acc_sh.at[pl.ds(s * ZR, ZR)])

    pltpu.sync_copy(src_hbm.at[pl.ds(start_row * CH, MAXR * CH)], srci_v)
    pltpu.sync_copy(dst2d_hbm.at[pl.ds(start_row, MAXR)], dsti_v)
    plsc.subcore_barrier()

    pltpu.async_copy(hp_sh.at[srci_v.at[pl.ds(0, CH)]], rows0_v, sem0)

    def body(g, carry):
        j0 = 2 * g
        j1 = j0 + 1
        cp1 = pltpu.async_copy(
            hp_sh.at[srci_v.at[pl.ds(j1 * CH, CH)]], rows1_v, sem1)
        pltpu.make_async_copy(hp_hbm.at[pl.ds(0, CH)], rows0_v, sem0).wait()
        pltpu.sync_copy(rows0_v, acc_sh.at[dsti_v.at[j0]], add=True)
        nxt = j0 + 2

        @pl.when(nxt < MAXR)
        def _issue_next_even():
            pltpu.async_copy(
                hp_sh.at[srci_v.at[pl.ds(nxt * CH, CH)]], rows0_v, sem0)

        cp1.wait()
        pltpu.sync_copy(rows1_v, acc_sh.at[dsti_v.at[j1]], add=True)
        return carry

    lax.fori_loop(0, MAXR // 2, body, 0)
    plsc.subcore_barrier()

    @pl.when(s == 0)
    def _flush():
        pltpu.sync_copy(acc_sh.at[pl.ds(0, N)], out_hbm.at[c])


_RB = 200


def _fold_body(b_ref, a1w_ref, a1b_ref, a2w_ref, a2b_ref, a3w_ref, a3b_ref,
               a4w_ref, a4b_ref, bae1_ref, bout_ref):
    v1 = jnp.dot(b_ref[...], a1w_ref[...], preferred_element_type=jnp.float32)
    v1 = v1 + a1b_ref[...]
    bae1_ref[...] = v1
    u = jnp.dot(v1, a2w_ref[...], preferred_element_type=jnp.float32) + a2b_ref[...]
    u = jnp.dot(u, a3w_ref[...], preferred_element_type=jnp.float32) + a3b_ref[...]
    bout_ref[...] = (
        jnp.dot(u, a4w_ref[...], preferred_element_type=jnp.float32) + a4b_ref[...]
    )


def _fold(b, A1W, A1b, A2W, A2b, A3W, A3b, A4W, A4b):
    full = lambda shape: pl.BlockSpec(shape, lambda i: (0, 0))
    return pl.pallas_call(
        _fold_body,
        grid=(N // _RB,),
        in_specs=[
            pl.BlockSpec((_RB, N), lambda i: (i, 0)),
            full((N, H)), full((1, H)),
            full((H, H)), full((1, H)),
            full((H, H)), full((1, H)),
            full((H, N)), full((1, N)),
        ],
        out_specs=[
            pl.BlockSpec((_RB, H), lambda i: (i, 0)),
            pl.BlockSpec((_RB, N), lambda i: (i, 0)),
        ],
        out_shape=[
            jax.ShapeDtypeStruct((N, H), jnp.float32),
            jax.ShapeDtypeStruct((N, N), jnp.float32),
        ],
        compiler_params=pltpu.CompilerParams(
            dimension_semantics=("parallel",),
        ),
    )(b, A1W, A1b.reshape(1, H), A2W, A2b.reshape(1, H),
      A3W, A3b.reshape(1, H), A4W, A4b.reshape(1, N))


def _glue1_body(x_ref, w1_ref, degc_ref, hp1_ref):
    dinv = lax.rsqrt(degc_ref[...] + 1.0)
    hp1_ref[...] = (
        jnp.dot(x_ref[...], w1_ref[...], preferred_element_type=jnp.float32) * dinv
    )


def _glue1(x, W1, degcol):
    return pl.pallas_call(
        _glue1_body,
        out_shape=jax.ShapeDtypeStruct((N, H), jnp.float32),
    )(x, W1, degcol)


def _bn_relu(t, g, be):
    m = jnp.mean(t, axis=0, keepdims=True)
    v = jnp.mean((t - m) ** 2, axis=0, keepdims=True)
    return jnp.maximum((t - m) * lax.rsqrt(v + 1e-5) * g + be, 0.0)


def _glue2_body(a_ref, b_ref, hp1_ref, degc_ref, b1_ref, g1_ref, be1_ref,
                bae1_ref, w2_ref, x0_ref, hp2_ref):
    dinv = lax.rsqrt(degc_ref[...] + 1.0)
    t = dinv * (a_ref[...] + b_ref[...] + hp1_ref[...]) + b1_ref[...]
    x0 = _bn_relu(t, g1_ref[...], be1_ref[...])
    x0_ref[...] = x0
    hp2_ref[...] = (
        jnp.dot(x0 + bae1_ref[...], w2_ref[...], preferred_element_type=jnp.float32)
        * dinv
    )


def _glue2(agg_a, agg_b, hp1, degcol, b1, g1, be1, bae1, W2):
    return pl.pallas_call(
        _glue2_body,
        out_shape=[
            jax.ShapeDtypeStruct((N, H), jnp.float32),
            jax.ShapeDtypeStruct((N, H), jnp.float32),
        ],
    )(agg_a, agg_b, hp1, degcol, b1.reshape(1, H), g1.reshape(1, H),
      be1.reshape(1, H), bae1, W2)


def _glue3_body(a_ref, b_ref, hp2_ref, degc_ref, b2_ref, g2_ref, be2_ref,
                x0_ref, jwa_ref, jwb_ref, jb_ref, out_ref):
    dinv = lax.rsqrt(degc_ref[...] + 1.0)
    t = dinv * (a_ref[...] + b_ref[...] + hp2_ref[...]) + b2_ref[...]
    x1 = _bn_relu(t, g2_ref[...], be2_ref[...])
    logits = (
        jnp.dot(x0_ref[...], jwa_ref[...], preferred_element_type=jnp.float32)
        + jnp.dot(x1, jwb_ref[...], preferred_element_type=jnp.float32)
        + jb_ref[...]
    )
    mx = jnp.max(logits, axis=1, keepdims=True)
    sh = logits - mx
    out_ref[...] = sh - jnp.log(jnp.sum(jnp.exp(sh), axis=1, keepdims=True))


def _glue3(agg_a, agg_b, hp2, degcol, b2, g2, be2, x0, JW, Jb):
    return pl.pallas_call(
        _glue3_body,
        out_shape=jax.ShapeDtypeStruct((N, C), jnp.float32),
    )(agg_a, agg_b, hp2, degcol, b2.reshape(1, H), g2.reshape(1, H),
      be2.reshape(1, H), x0, JW[:H], JW[H:], Jb.reshape(1, C))


def _foldA_body(b_ref, a1w_ref, a1b_ref, bae1_ref):
    bae1_ref[...] = (
        jnp.dot(b_ref[...], a1w_ref[...], preferred_element_type=jnp.float32)
        + a1b_ref[...]
    )


def _foldB_body(bae1_ref, a2w_ref, a2b_ref, a3w_ref, a3b_ref,
                a4w_ref, a4b_ref, bout_ref):
    u = jnp.dot(bae1_ref[...], a2w_ref[...],
                preferred_element_type=jnp.float32) + a2b_ref[...]
    u = jnp.dot(u, a3w_ref[...], preferred_element_type=jnp.float32) + a3b_ref[...]
    bout_ref[...] = (
        jnp.dot(u, a4w_ref[...], preferred_element_type=jnp.float32) + a4b_ref[...]
    )


def _fold_split(b, A1W, A1b, A2W, A2b, A3W, A3b, A4W, A4b):
    full = lambda shape: pl.BlockSpec(shape, lambda i: (0, 0))
    bae1 = pl.pallas_call(
        _foldA_body,
        grid=(N // _RB,),
        in_specs=[
            pl.BlockSpec((_RB, N), lambda i: (i, 0)),
            full((N, H)), full((1, H)),
        ],
        out_specs=pl.BlockSpec((_RB, H), lambda i: (i, 0)),
        out_shape=jax.ShapeDtypeStruct((N, H), jnp.float32),
        compiler_params=pltpu.CompilerParams(
            dimension_semantics=("parallel",),
        ),
    )(b, A1W, A1b.reshape(1, H))
    bout = pl.pallas_call(
        _foldB_body,
        grid=(N // _RB,),
        in_specs=[
            pl.BlockSpec((_RB, H), lambda i: (i, 0)),
            full((H, H)), full((1, H)),
            full((H, H)), full((1, H)),
            full((H, N)), full((1, N)),
        ],
        out_specs=pl.BlockSpec((_RB, N), lambda i: (i, 0)),
        out_shape=jax.ShapeDtypeStruct((N, N), jnp.float32),
        compiler_params=pltpu.CompilerParams(
            dimension_semantics=("parallel",),
        ),
    )(bae1, A2W, A2b.reshape(1, H), A3W, A3b.reshape(1, H),
      A4W, A4b.reshape(1, N))
    return bae1, bout


def kernel(b, x, edge_index, W1, b1, W2, b2, g1, be1, g2, be2,
           A1W, A1b, A2W, A2b, A3W, A3b, A4W, A4b, JW, Jb):
    src = edge_index[0]
    dst = edge_index[1]
    pad = PAD_ROWS * CH - E
    src_pad = jnp.concatenate([src, jnp.zeros((pad,), jnp.int32)])
    dpad = N + (jnp.arange(pad, dtype=jnp.int32) % 8)
    dst2d = jnp.concatenate([dst, dpad]).reshape(PAD_ROWS, CH)
    zeros_n = jnp.zeros((NPD,), jnp.float32)
    zeros_nh = jnp.zeros((NP, H), jnp.float32)
    ones_ch = jnp.ones((CH,), jnp.float32)

    deg2 = _deg_sc(dst2d, ones_ch, zeros_n)
    degcol = (deg2[0, :N] + deg2[1, :N]).reshape(N, 1)

    hp1 = _glue1(x, W1, degcol)
    agg1 = _agg_sc(hp1, src_pad, dst2d, zeros_nh)

    bae1, bout = _fold_split(b, A1W, A1b, A2W, A2b, A3W, A3b, A4W, A4b)

    x0, hp2 = _glue2(agg1[0], agg1[1], hp1, degcol, b1, g1, be1, bae1, W2)
    agg2 = _agg_sc(hp2, src_pad, dst2d, zeros_nh)
    out1 = _glue3(agg2[0], agg2[1], hp2, degcol, b2, g2, be2, x0, JW, Jb)
    return out1, bout


# speedup vs baseline: 13.0448x; 1.0026x over previous
import functools

import jax
import jax.numpy as jnp
from jax import lax
from jax.experimental import pallas as pl
from jax.experimental.pallas import tpu as pltpu
from jax.experimental.pallas import tpu_sc as plsc

N = 10000
E = 160000
F_IN = 128
H = 32
C = 40

NC = 2
NS = 16
NW = NC * NS
CH = 256
NROWS = E // CH
MAXR = (NROWS + NW - 1) // NW
PAD_ROWS = MAXR * NW
NP = N + 16
NPD = 10240

_MESH = plsc.VectorSubcoreMesh(core_axis_name="c", subcore_axis_name="s")


@functools.partial(
    pl.kernel,
    out_type=jax.ShapeDtypeStruct((NC, NPD), jnp.float32),
    mesh=_MESH,
    scratch_types=[
        pltpu.VMEM((MAXR, CH), jnp.int32),
        pltpu.VMEM((CH,), jnp.float32),
        pltpu.VMEM_SHARED((NPD,), jnp.float32),
        pltpu.SemaphoreType.DMA,
    ],
    compiler_params=pltpu.CompilerParams(use_tc_tiling_on_sc=False),
)
def _deg_sc(dst2d_hbm, ones_hbm, zeros_hbm, out_hbm, dsti_v, ones_v, acc_sh, sem):
    c = lax.axis_index("c")
    s = lax.axis_index("s")
    w = c * NS + s
    start_row = w * MAXR

    @pl.when(s == 0)
    def _zero():
        pltpu.sync_copy(zeros_hbm, acc_sh)

    pltpu.sync_copy(dst2d_hbm.at[pl.ds(start_row, MAXR)], dsti_v)
    pltpu.sync_copy(ones_hbm, ones_v)
    plsc.subcore_barrier()

    def body(j, carry):
        pltpu.sync_copy(ones_v, acc_sh.at[dsti_v.at[j]], add=True)
        return carry

    lax.fori_loop(0, MAXR, body, 0)
    plsc.subcore_barrier()

    @pl.when(s == 0)
    def _flush():
        pltpu.sync_copy(acc_sh, out_hbm.at[c])


@functools.partial(
    pl.kernel,
    out_type=jax.ShapeDtypeStruct((NC, N, H), jnp.float32),
    mesh=_MESH,
    scratch_types=[
        pltpu.VMEM((MAXR * CH,), jnp.int32),
        pltpu.VMEM((MAXR, CH), jnp.int32),
        pltpu.VMEM((CH, H), jnp.float32),
        pltpu.VMEM((CH, H), jnp.float32),
        pltpu.VMEM_SHARED((N, H), jnp.float32),
        pltpu.VMEM_SHARED((NP, H), jnp.float32),
        pltpu.SemaphoreType.DMA,
        pltpu.SemaphoreType.DMA,
    ],
    compiler_params=pltpu.CompilerParams(use_tc_tiling_on_sc=False),
)
def _agg_sc(hp_hbm, src_hbm, dst2d_hbm, zeros_hbm, out_hbm,
            srci_v, dsti_v, rows0_v, rows1_v, hp_sh, acc_sh, sem0, sem1):
    c = lax.axis_index("c")
    s = lax.axis_index("s")
    w = c * NS + s
    start_row = w * MAXR

    TR = N // NS
    ZR = NP // NS
    pltpu.sync_copy(hp_hbm.at[pl.ds(s * TR, TR)], hp_sh.at[pl.ds(s * TR, TR)])
    pltpu.sync_copy(zeros_hbm.at[pl.ds(s * ZR, ZR)], acc_sh.at[pl.ds(s * ZR, ZR)])

    pltpu.sync_copy(src_hbm.at[pl.ds(start_row * CH, MAXR * CH)], srci_v)
    pltpu.sync_copy(dst2d_hbm.at[pl.ds(start_row, MAXR)], dsti_v)
    plsc.subcore_barrier()

    pltpu.async_copy(hp_sh.at[srci_v.at[pl.ds(0, CH)]], rows0_v, sem0)

    def body(g, carry):
        j0 = 2 * g
        j1 = j0 + 1
        cp1 = pltpu.async_copy(
            hp_sh.at[srci_v.at[pl.ds(j1 * CH, CH)]], rows1_v, sem1)
        pltpu.make_async_copy(hp_hbm.at[pl.ds(0, CH)], rows0_v, sem0).wait()
        pltpu.sync_copy(rows0_v, acc_sh.at[dsti_v.at[j0]], add=True)
        nxt = j0 + 2

        @pl.when(nxt < MAXR)
        def _issue_next_even():
            pltpu.async_copy(
                hp_sh.at[srci_v.at[pl.ds(nxt * CH, CH)]], rows0_v, sem0)

        cp1.wait()
        pltpu.sync_copy(rows1_v, acc_sh.at[dsti_v.at[j1]], add=True)
        return carry

    lax.fori_loop(0, MAXR // 2, body, 0)
    plsc.subcore_barrier()

    @pl.when(s == 0)
    def _flush():
        pltpu.sync_copy(acc_sh.at[pl.ds(0, N)], out_hbm.at[c])


_RB = 200


def _fold_body(b_ref, a1w_ref, a1b_ref, a2w_ref, a2b_ref, a3w_ref, a3b_ref,
               a4w_ref, a4b_ref, bae1_ref, bout_ref):
    v1 = jnp.dot(b_ref[...], a1w_ref[...], preferred_element_type=jnp.float32)
    v1 = v1 + a1b_ref[...]
    bae1_ref[...] = v1
    u = jnp.dot(v1, a2w_ref[...], preferred_element_type=jnp.float32) + a2b_ref[...]
    u = jnp.dot(u, a3w_ref[...], preferred_element_type=jnp.float32) + a3b_ref[...]
    bout_ref[...] = (
        jnp.dot(u, a4w_ref[...], preferred_element_type=jnp.float32) + a4b_ref[...]
    )


def _fold(b, A1W, A1b, A2W, A2b, A3W, A3b, A4W, A4b):
    full = lambda shape: pl.BlockSpec(shape, lambda i: (0, 0))
    return pl.pallas_call(
        _fold_body,
        grid=(N // _RB,),
        in_specs=[
            pl.BlockSpec((_RB, N), lambda i: (i, 0)),
            full((N, H)), full((1, H)),
            full((H, H)), full((1, H)),
            full((H, H)), full((1, H)),
            full((H, N)), full((1, N)),
        ],
        out_specs=[
            pl.BlockSpec((_RB, H), lambda i: (i, 0)),
            pl.BlockSpec((_RB, N), lambda i: (i, 0)),
        ],
        out_shape=[
            jax.ShapeDtypeStruct((N, H), jnp.float32),
            jax.ShapeDtypeStruct((N, N), jnp.float32),
        ],
        compiler_params=pltpu.CompilerParams(
            dimension_semantics=("parallel",),
        ),
    )(b, A1W, A1b.reshape(1, H), A2W, A2b.reshape(1, H),
      A3W, A3b.reshape(1, H), A4W, A4b.reshape(1, N))


def _glue1_body(x_ref, w1_ref, degc_ref, hp1_ref):
    dinv = lax.rsqrt(degc_ref[...] + 1.0)
    hp1_ref[...] = (
        jnp.dot(x_ref[...], w1_ref[...], preferred_element_type=jnp.float32) * dinv
    )


def _glue1(x, W1, degcol):
    return pl.pallas_call(
        _glue1_body,
        out_shape=jax.ShapeDtypeStruct((N, H), jnp.float32),
    )(x, W1, degcol)


def _bn_relu(t, g, be):
    m = jnp.mean(t, axis=0, keepdims=True)
    v = jnp.mean((t - m) ** 2, axis=0, keepdims=True)
    return jnp.maximum((t - m) * lax.rsqrt(v + 1e-5) * g + be, 0.0)


def _glue2_body(a_ref, b_ref, hp1_ref, degc_ref, b1_ref, g1_ref, be1_ref,
                bae1_ref, w2_ref, x0_ref, hp2_ref):
    dinv = lax.rsqrt(degc_ref[...] + 1.0)
    t = dinv * (a_ref[...] + b_ref[...] + hp1_ref[...]) + b1_ref[...]
    x0 = _bn_relu(t, g1_ref[...], be1_ref[...])
    x0_ref[...] = x0
    hp2_ref[...] = (
        jnp.dot(x0 + bae1_ref[...], w2_ref[...], preferred_element_type=jnp.float32)
        * dinv
    )


def _glue2(agg_a, agg_b, hp1, degcol, b1, g1, be1, bae1, W2):
    return pl.pallas_call(
        _glue2_body,
        out_shape=[
            jax.ShapeDtypeStruct((N, H), jnp.float32),
            jax.ShapeDtypeStruct((N, H), jnp.float32),
        ],
    )(agg_a, agg_b, hp1, degcol, b1.reshape(1, H), g1.reshape(1, H),
      be1.reshape(1, H), bae1, W2)


def _glue3_body(a_ref, b_ref, hp2_ref, degc_ref, b2_ref, g2_ref, be2_ref,
                x0_ref, jwa_ref, jwb_ref, jb_ref, out_ref):
    dinv = lax.rsqrt(degc_ref[...] + 1.0)
    t = dinv * (a_ref[...] + b_ref[...] + hp2_ref[...]) + b2_ref[...]
    x1 = _bn_relu(t, g2_ref[...], be2_ref[...])
    logits = (
        jnp.dot(x0_ref[...], jwa_ref[...], preferred_element_type=jnp.float32)
        + jnp.dot(x1, jwb_ref[...], preferred_element_type=jnp.float32)
        + jb_ref[...]
    )
    mx = jnp.max(logits, axis=1, keepdims=True)
    sh = logits - mx
    out_ref[...] = sh - jnp.log(jnp.sum(jnp.exp(sh), axis=1, keepdims=True))


def _glue3(agg_a, agg_b, hp2, degcol, b2, g2, be2, x0, JW, Jb):
    return pl.pallas_call(
        _glue3_body,
        out_shape=jax.ShapeDtypeStruct((N, C), jnp.float32),
    )(agg_a, agg_b, hp2, degcol, b2.reshape(1, H), g2.reshape(1, H),
      be2.reshape(1, H), x0, JW[:H], JW[H:], Jb.reshape(1, C))


def _foldA_body(b_ref, a1w_ref, a1b_ref, bae1_ref):
    bae1_ref[...] = (
        jnp.dot(b_ref[...], a1w_ref[...], preferred_element_type=jnp.float32)
        + a1b_ref[...]
    )


def _foldB_body(bae1_ref, a2w_ref, a2b_ref, a3w_ref, a3b_ref,
                a4w_ref, a4b_ref, bout_ref):
    u = jnp.dot(bae1_ref[...], a2w_ref[...],
                preferred_element_type=jnp.float32) + a2b_ref[...]
    u = jnp.dot(u, a3w_ref[...], preferred_element_type=jnp.float32) + a3b_ref[...]
    bout_ref[...] = (
        jnp.dot(u, a4w_ref[...], preferred_element_type=jnp.float32) + a4b_ref[...]
    )


def _fold_split(b, A1W, A1b, A2W, A2b, A3W, A3b, A4W, A4b):
    full = lambda shape: pl.BlockSpec(shape, lambda i: (0, 0))
    bae1 = pl.pallas_call(
        _foldA_body,
        grid=(N // _RB,),
        in_specs=[
            pl.BlockSpec((_RB, N), lambda i: (i, 0)),
            full((N, H)), full((1, H)),
        ],
        out_specs=pl.BlockSpec((_RB, H), lambda i: (i, 0)),
        out_shape=jax.ShapeDtypeStruct((N, H), jnp.float32),
        compiler_params=pltpu.CompilerParams(
            dimension_semantics=("parallel",),
        ),
    )(b, A1W, A1b.reshape(1, H))
    bout = pl.pallas_call(
        _foldB_body,
        grid=(N // _RB,),
        in_specs=[
            pl.BlockSpec((_RB, H), lambda i: (i, 0)),
            full((H, H)), full((1, H)),
            full((H, H)), full((1, H)),
            full((H, N)), full((1, N)),
        ],
        out_specs=pl.BlockSpec((_RB, N), lambda i: (i, 0)),
        out_shape=jax.ShapeDtypeStruct((N, N), jnp.float32),
        compiler_params=pltpu.CompilerParams(
            dimension_semantics=("parallel",),
        ),
    )(bae1, A2W, A2b.reshape(1, H), A3W, A3b.reshape(1, H),
      A4W, A4b.reshape(1, N))
    return bae1, bout


def kernel(b, x, edge_index, W1, b1, W2, b2, g1, be1, g2, be2,
           A1W, A1b, A2W, A2b, A3W, A3b, A4W, A4b, JW, Jb):
    src = edge_index[0]
    dst = edge_index[1]
    pad = PAD_ROWS * CH - E
    src_pad = jnp.concatenate([src, jnp.zeros((pad,), jnp.int32)])
    dpad = N + (jnp.arange(pad, dtype=jnp.int32) % 8)
    dst2d = jnp.concatenate([dst, dpad]).reshape(PAD_ROWS, CH)
    zeros_n = jnp.zeros((NPD,), jnp.float32)
    zeros_nh = jnp.zeros((NP, H), jnp.float32)
    ones_ch = jnp.ones((CH,), jnp.float32)

    deg2 = _deg_sc(dst2d, ones_ch, zeros_n)
    degcol = (deg2[0, :N] + deg2[1, :N]).reshape(N, 1)

    hp1 = _glue1(x, W1, degcol)
    agg1 = _agg_sc(hp1, src_pad, dst2d, zeros_nh)

    bae1, bout = _fold_split(b, A1W, A1b, A2W, A2b, A3W, A3b, A4W, A4b)

    x0, hp2 = _glue2(agg1[0], agg1[1], hp1, degcol, b1, g1, be1, bae1, W2)
    agg2 = _agg_sc(hp2, src_pad, dst2d, zeros_nh)
    out1 = _glue3(agg2[0], agg2[1], hp2, degcol, b2, g2, be2, x0, JW, Jb)
    return out1, bout


# speedup vs baseline: 13.0533x; 1.0007x over previous
import functools

import jax
import jax.numpy as jnp
from jax import lax
from jax.experimental import pallas as pl
from jax.experimental.pallas import tpu as pltpu
from jax.experimental.pallas import tpu_sc as plsc

N = 10000
E = 160000
F_IN = 128
H = 32
C = 40

NC = 2
NS = 16
NW = NC * NS
CH = 200
NROWS = E // CH
MAXR = NROWS // NW
NP = N + 16
NPD = 10240

_MESH = plsc.VectorSubcoreMesh(core_axis_name="c", subcore_axis_name="s")


@functools.partial(
    pl.kernel,
    out_type=jax.ShapeDtypeStruct((NC, NPD), jnp.float32),
    mesh=_MESH,
    scratch_types=[
        pltpu.VMEM((MAXR, CH), jnp.int32),
        pltpu.VMEM((CH,), jnp.float32),
        pltpu.VMEM_SHARED((NPD,), jnp.float32),
        pltpu.SemaphoreType.DMA,
    ],
    compiler_params=pltpu.CompilerParams(use_tc_tiling_on_sc=False),
)
def _deg_sc(dst2d_hbm, ones_hbm, zeros_hbm, out_hbm, dsti_v, ones_v, acc_sh, sem):
    c = lax.axis_index("c")
    s = lax.axis_index("s")
    w = c * NS + s
    start_row = w * MAXR

    @pl.when(s == 0)
    def _zero():
        pltpu.sync_copy(zeros_hbm, acc_sh)

    pltpu.sync_copy(dst2d_hbm.at[pl.ds(start_row, MAXR)], dsti_v)
    pltpu.sync_copy(ones_hbm, ones_v)
    plsc.subcore_barrier()

    def body(j, carry):
        pltpu.sync_copy(ones_v, acc_sh.at[dsti_v.at[j]], add=True)
        return carry

    lax.fori_loop(0, MAXR, body, 0)
    plsc.subcore_barrier()

    @pl.when(s == 0)
    def _flush():
        pltpu.sync_copy(acc_sh, out_hbm.at[c])


@functools.partial(
    pl.kernel,
    out_type=jax.ShapeDtypeStruct((NC, N, H), jnp.float32),
    mesh=_MESH,
    scratch_types=[
        pltpu.VMEM((MAXR * CH,), jnp.int32),
        pltpu.VMEM((MAXR, CH), jnp.int32),
        pltpu.VMEM((CH, H), jnp.float32),
        pltpu.VMEM((CH, H), jnp.float32),
        pltpu.VMEM_SHARED((N, H), jnp.float32),
        pltpu.VMEM_SHARED((NP, H), jnp.float32),
        pltpu.SemaphoreType.DMA,
        pltpu.SemaphoreType.DMA,
    ],
    compiler_params=pltpu.CompilerParams(use_tc_tiling_on_sc=False),
)
def _agg_sc(hp_hbm, src_hbm, dst2d_hbm, zeros_hbm, out_hbm,
            srci_v, dsti_v, rows0_v, rows1_v, hp_sh, acc_sh, sem0, sem1):
    c = lax.axis_index("c")
    s = lax.axis_index("s")
    w = c * NS + s
    start_row = w * MAXR

    TR = N // NS
    ZR = NP // NS
    pltpu.sync_copy(hp_hbm.at[pl.ds(s * TR, TR)], hp_sh.at[pl.ds(s * TR, TR)])
    pltpu.sync_copy(zeros_hbm.at[pl.ds(s * ZR, ZR)], acc_sh.at[pl.ds(s * ZR, ZR)])

    pltpu.sync_copy(src_hbm.at[pl.ds(start_row * CH, MAXR * CH)], srci_v)
    pltpu.sync_copy(dst2d_hbm.at[pl.ds(start_row, MAXR)], dsti_v)
    plsc.subcore_barrier()

    pltpu.async_copy(hp_sh.at[srci_v.at[pl.ds(0, CH)]], rows0_v, sem0)

    def body(g, carry):
        j0 = 2 * g
        j1 = j0 + 1
        cp1 = pltpu.async_copy(
            hp_sh.at[srci_v.at[pl.ds(j1 * CH, CH)]], rows1_v, sem1)
        pltpu.make_async_copy(hp_hbm.at[pl.ds(0, CH)], rows0_v, sem0).wait()
        pltpu.sync_copy(rows0_v, acc_sh.at[dsti_v.at[j0]], add=True)
        nxt = j0 + 2

        @pl.when(nxt < MAXR)
        def _issue_next_even():
            pltpu.async_copy(
                hp_sh.at[srci_v.at[pl.ds(nxt * CH, CH)]], rows0_v, sem0)

        cp1.wait()
        pltpu.sync_copy(rows1_v, acc_sh.at[dsti_v.at[j1]], add=True)
        return carry

    lax.fori_loop(0, MAXR // 2, body, 0)
    pltpu.make_async_copy(hp_hbm.at[pl.ds(0, CH)], rows0_v, sem0).wait()
    pltpu.sync_copy(rows0_v, acc_sh.at[dsti_v.at[MAXR - 1]], add=True)
    plsc.subcore_barrier()

    @pl.when(s == 0)
    def _flush():
        pltpu.sync_copy(acc_sh.at[pl.ds(0, N)], out_hbm.at[c])


_RB = 200


def _fold_body(b_ref, a1w_ref, a1b_ref, a2w_ref, a2b_ref, a3w_ref, a3b_ref,
               a4w_ref, a4b_ref, bae1_ref, bout_ref):
    v1 = jnp.dot(b_ref[...], a1w_ref[...], preferred_element_type=jnp.float32)
    v1 = v1 + a1b_ref[...]
    bae1_ref[...] = v1
    u = jnp.dot(v1, a2w_ref[...], preferred_element_type=jnp.float32) + a2b_ref[...]
    u = jnp.dot(u, a3w_ref[...], preferred_element_type=jnp.float32) + a3b_ref[...]
    bout_ref[...] = (
        jnp.dot(u, a4w_ref[...], preferred_element_type=jnp.float32) + a4b_ref[...]
    )


def _fold(b, A1W, A1b, A2W, A2b, A3W, A3b, A4W, A4b):
    full = lambda shape: pl.BlockSpec(shape, lambda i: (0, 0))
    return pl.pallas_call(
        _fold_body,
        grid=(N // _RB,),
        in_specs=[
            pl.BlockSpec((_RB, N), lambda i: (i, 0)),
            full((N, H)), full((1, H)),
            full((H, H)), full((1, H)),
            full((H, H)), full((1, H)),
            full((H, N)), full((1, N)),
        ],
        out_specs=[
            pl.BlockSpec((_RB, H), lambda i: (i, 0)),
            pl.BlockSpec((_RB, N), lambda i: (i, 0)),
        ],
        out_shape=[
            jax.ShapeDtypeStruct((N, H), jnp.float32),
            jax.ShapeDtypeStruct((N, N), jnp.float32),
        ],
        compiler_params=pltpu.CompilerParams(
            dimension_semantics=("parallel",),
        ),
    )(b, A1W, A1b.reshape(1, H), A2W, A2b.reshape(1, H),
      A3W, A3b.reshape(1, H), A4W, A4b.reshape(1, N))


def _glue1_body(x_ref, w1_ref, degc_ref, hp1_ref):
    dinv = lax.rsqrt(degc_ref[...] + 1.0)
    hp1_ref[...] = (
        jnp.dot(x_ref[...], w1_ref[...], preferred_element_type=jnp.float32) * dinv
    )


def _glue1(x, W1, degcol):
    return pl.pallas_call(
        _glue1_body,
        out_shape=jax.ShapeDtypeStruct((N, H), jnp.float32),
    )(x, W1, degcol)


def _bn_relu(t, g, be):
    m = jnp.mean(t, axis=0, keepdims=True)
    v = jnp.mean((t - m) ** 2, axis=0, keepdims=True)
    return jnp.maximum((t - m) * lax.rsqrt(v + 1e-5) * g + be, 0.0)


def _glue2_body(a_ref, b_ref, hp1_ref, degc_ref, b1_ref, g1_ref, be1_ref,
                bae1_ref, w2_ref, x0_ref, hp2_ref):
    dinv = lax.rsqrt(degc_ref[...] + 1.0)
    t = dinv * (a_ref[...] + b_ref[...] + hp1_ref[...]) + b1_ref[...]
    x0 = _bn_relu(t, g1_ref[...], be1_ref[...])
    x0_ref[...] = x0
    hp2_ref[...] = (
        jnp.dot(x0 + bae1_ref[...], w2_ref[...], preferred_element_type=jnp.float32)
        * dinv
    )


def _glue2(agg_a, agg_b, hp1, degcol, b1, g1, be1, bae1, W2):
    return pl.pallas_call(
        _glue2_body,
        out_shape=[
            jax.ShapeDtypeStruct((N, H), jnp.float32),
            jax.ShapeDtypeStruct((N, H), jnp.float32),
        ],
    )(agg_a, agg_b, hp1, degcol, b1.reshape(1, H), g1.reshape(1, H),
      be1.reshape(1, H), bae1, W2)


def _glue3_body(a_ref, b_ref, hp2_ref, degc_ref, b2_ref, g2_ref, be2_ref,
                x0_ref, jwa_ref, jwb_ref, jb_ref, out_ref):
    dinv = lax.rsqrt(degc_ref[...] + 1.0)
    t = dinv * (a_ref[...] + b_ref[...] + hp2_ref[...]) + b2_ref[...]
    x1 = _bn_relu(t, g2_ref[...], be2_ref[...])
    logits = (
        jnp.dot(x0_ref[...], jwa_ref[...], preferred_element_type=jnp.float32)
        + jnp.dot(x1, jwb_ref[...], preferred_element_type=jnp.float32)
        + jb_ref[...]
    )
    mx = jnp.max(logits, axis=1, keepdims=True)
    sh = logits - mx
    out_ref[...] = sh - jnp.log(jnp.sum(jnp.exp(sh), axis=1, keepdims=True))


def _glue3(agg_a, agg_b, hp2, degcol, b2, g2, be2, x0, JW, Jb):
    return pl.pallas_call(
        _glue3_body,
        out_shape=jax.ShapeDtypeStruct((N, C), jnp.float32),
    )(agg_a, agg_b, hp2, degcol, b2.reshape(1, H), g2.reshape(1, H),
      be2.reshape(1, H), x0, JW[:H], JW[H:], Jb.reshape(1, C))


def _foldA_body(b_ref, a1w_ref, a1b_ref, bae1_ref):
    bae1_ref[...] = (
        jnp.dot(b_ref[...], a1w_ref[...], preferred_element_type=jnp.float32)
        + a1b_ref[...]
    )


def _foldB_body(bae1_ref, a2w_ref, a2b_ref, a3w_ref, a3b_ref,
                a4w_ref, a4b_ref, bout_ref):
    u = jnp.dot(bae1_ref[...], a2w_ref[...],
                preferred_element_type=jnp.float32) + a2b_ref[...]
    u = jnp.dot(u, a3w_ref[...], preferred_element_type=jnp.float32) + a3b_ref[...]
    bout_ref[...] = (
        jnp.dot(u, a4w_ref[...], preferred_element_type=jnp.float32) + a4b_ref[...]
    )


def _fold_split(b, A1W, A1b, A2W, A2b, A3W, A3b, A4W, A4b):
    full = lambda shape: pl.BlockSpec(shape, lambda i: (0, 0))
    bae1 = pl.pallas_call(
        _foldA_body,
        grid=(N // _RB,),
        in_specs=[
            pl.BlockSpec((_RB, N), lambda i: (i, 0)),
            full((N, H)), full((1, H)),
        ],
        out_specs=pl.BlockSpec((_RB, H), lambda i: (i, 0)),
        out_shape=jax.ShapeDtypeStruct((N, H), jnp.float32),
        compiler_params=pltpu.CompilerParams(
            dimension_semantics=("parallel",),
        ),
    )(b, A1W, A1b.reshape(1, H))
    bout = pl.pallas_call(
        _foldB_body,
        grid=(N // _RB,),
        in_specs=[
            pl.BlockSpec((_RB, H), lambda i: (i, 0)),
            full((H, H)), full((1, H)),
            full((H, H)), full((1, H)),
            full((H, N)), full((1, N)),
        ],
        out_specs=pl.BlockSpec((_RB, N), lambda i: (i, 0)),
        out_shape=jax.ShapeDtypeStruct((N, N), jnp.float32),
        compiler_params=pltpu.CompilerParams(
            dimension_semantics=("parallel",),
        ),
    )(bae1, A2W, A2b.reshape(1, H), A3W, A3b.reshape(1, H),
      A4W, A4b.reshape(1, N))
    return bae1, bout


def kernel(b, x, edge_index, W1, b1, W2, b2, g1, be1, g2, be2,
           A1W, A1b, A2W, A2b, A3W, A3b, A4W, A4b, JW, Jb):
    src = edge_index[0]
    dst = edge_index[1]
    src_pad = src
    dst2d = dst.reshape(NROWS, CH)
    zeros_n = jnp.zeros((NPD,), jnp.float32)
    zeros_nh = jnp.zeros((NP, H), jnp.float32)
    ones_ch = jnp.ones((CH,), jnp.float32)

    deg2 = _deg_sc(dst2d, ones_ch, zeros_n)
    degcol = (deg2[0, :N] + deg2[1, :N]).reshape(N, 1)

    hp1 = _glue1(x, W1, degcol)
    agg1 = _agg_sc(hp1, src_pad, dst2d, zeros_nh)

    bae1, bout = _fold_split(b, A1W, A1b, A2W, A2b, A3W, A3b, A4W, A4b)

    x0, hp2 = _glue2(agg1[0], agg1[1], hp1, degcol, b1, g1, be1, bae1, W2)
    agg2 = _agg_sc(hp2, src_pad, dst2d, zeros_nh)
    out1 = _glue3(agg2[0], agg2[1], hp2, degcol, b2, g2, be2, x0, JW, Jb)
    return out1, bout
